# baseline (speedup 1.0000x reference)
import functools
import jax
import jax.numpy as jnp
from jax import lax
from jax.experimental import pallas as pl
from jax.experimental.pallas import tpu as pltpu
from jax.experimental.pallas import tpu_sc as plsc


_ROWS = 16384
_K = 128
_TC_ROWS = 12288
_SC_ROWS = _ROWS - _TC_ROWS

_BLOCK_R = 1024

_NC = 2
_NS = 16
_NW = _NC * _NS
_SC_ROWS_PER_W = _SC_ROWS // _NW
_CHUNK_ROWS = 64
_NCHUNK = _SC_ROWS_PER_W // _CHUNK_ROWS


def _select_body(x_ref, s_ref, o_ref):
    o_ref[...] = jnp.dot(
        x_ref[...], s_ref[...], preferred_element_type=jnp.float32
    )


def _sc_body(x_hbm, idx_hbm, out_hbm, idx_v, data_v, sem):
    wid = lax.axis_index("s") * _NC + lax.axis_index("c")
    e0 = wid * _SC_ROWS_PER_W * _K

    def chunk(ci, _):
        base = e0 + ci * _CHUNK_ROWS * _K
        pltpu.sync_copy(idx_hbm.at[pl.ds(base, _CHUNK_ROWS * _K)], idx_v)
        pltpu.async_copy(x_hbm.at[idx_v], data_v, sem).wait()
        pltpu.sync_copy(data_v, out_hbm.at[pl.ds(base, _CHUNK_ROWS * _K)])
        return ()

    lax.fori_loop(0, _NCHUNK, chunk, ())


def kernel(x, w):
    b, srows, cols = x.shape
    k = w.shape[0]
    xf = x.reshape(b * srows, cols)

    sel = (
        jax.lax.broadcasted_iota(jnp.int32, (cols, k), 0) == w[None, :]
    ).astype(jnp.float32)
    out_tc = pl.pallas_call(
        _select_body,
        grid=(_TC_ROWS // _BLOCK_R,),
        in_specs=[
            pl.BlockSpec((_BLOCK_R, cols), lambda i: (i, 0)),
            pl.BlockSpec((cols, k), lambda i: (0, 0)),
        ],
        out_specs=pl.BlockSpec((_BLOCK_R, k), lambda i: (i, 0)),
        out_shape=jax.ShapeDtypeStruct((_TC_ROWS, k), jnp.float32),
    )(xf[:_TC_ROWS], sel)

    idx = (
        (jnp.arange(_SC_ROWS, dtype=jnp.int32) + _TC_ROWS)[:, None] * cols
        + w[None, :]
    ).reshape(_SC_ROWS * k)
    mesh = plsc.VectorSubcoreMesh(core_axis_name="c", subcore_axis_name="s")
    sc_call = functools.partial(
        pl.kernel,
        mesh=mesh,
        out_type=jax.ShapeDtypeStruct((_SC_ROWS * k,), jnp.float32),
        scratch_types=[
            pltpu.VMEM((_CHUNK_ROWS * _K,), jnp.int32),
            pltpu.VMEM((_CHUNK_ROWS * _K,), jnp.float32),
            pltpu.SemaphoreType.DMA,
        ],
    )(_sc_body)
    out_sc = sc_call(x.reshape(b * srows * cols), idx).reshape(_SC_ROWS, k)

    out = jnp.concatenate([out_tc, out_sc], axis=0)
    return out.reshape(b, srows, k)


# speedup vs baseline: 1.4411x; 1.4411x over previous
import functools
import jax
import jax.numpy as jnp
from jax import lax
from jax.experimental import pallas as pl
from jax.experimental.pallas import tpu as pltpu
from jax.experimental.pallas import tpu_sc as plsc


_ROWS = 16384
_K = 128
_TC_ROWS = 12288
_SC_ROWS = _ROWS - _TC_ROWS

_BLOCK_R = 1024

_NC = 2
_NS = 16
_NW = _NC * _NS
_SC_ROWS_PER_W = _SC_ROWS // _NW
_CHUNK_ROWS = 64
_NCHUNK = _SC_ROWS_PER_W // _CHUNK_ROWS


def _select_body(x_ref, s_ref, o_ref):
    o_ref[...] = jnp.dot(
        x_ref[...], s_ref[...], preferred_element_type=jnp.float32
    )


def _sc_body(x_hbm, idx_hbm, out_hbm, idx_v, data_v, sem):
    wid = lax.axis_index("s") * _NC + lax.axis_index("c")
    e0 = wid * _SC_ROWS_PER_W * _K

    def chunk(ci, _):
        base = e0 + ci * _CHUNK_ROWS * _K
        pltpu.sync_copy(idx_hbm.at[pl.ds(base, _CHUNK_ROWS * _K)], idx_v)
        pltpu.async_copy(x_hbm.at[idx_v], data_v, sem).wait()
        pltpu.sync_copy(data_v, out_hbm.at[pl.ds(base, _CHUNK_ROWS * _K)])
        return ()

    lax.fori_loop(0, _NCHUNK, chunk, ())


def kernel(x, w):
    b, srows, cols = x.shape
    k = w.shape[0]
    xf = x.reshape(b * srows, cols)

    sel = (
        jax.lax.broadcasted_iota(jnp.int32, (cols, k), 0) == w[None, :]
    ).astype(jnp.float32)
    out_tc = pl.pallas_call(
        _select_body,
        grid=(_TC_ROWS // _BLOCK_R,),
        in_specs=[
            pl.BlockSpec((_BLOCK_R, cols), lambda i: (i, 0)),
            pl.BlockSpec((cols, k), lambda i: (0, 0)),
        ],
        out_specs=pl.BlockSpec((_BLOCK_R, k), lambda i: (i, 0)),
        out_shape=jax.ShapeDtypeStruct((_TC_ROWS, k), jnp.float32),
    )(xf, sel)

    idx = (
        (jnp.arange(_SC_ROWS, dtype=jnp.int32) + _TC_ROWS)[:, None] * cols
        + w[None, :]
    ).reshape(_SC_ROWS * k)
    mesh = plsc.VectorSubcoreMesh(core_axis_name="c", subcore_axis_name="s")
    sc_call = functools.partial(
        pl.kernel,
        mesh=mesh,
        out_type=jax.ShapeDtypeStruct((_SC_ROWS * k,), jnp.float32),
        scratch_types=[
            pltpu.VMEM((_CHUNK_ROWS * _K,), jnp.int32),
            pltpu.VMEM((_CHUNK_ROWS * _K,), jnp.float32),
            pltpu.SemaphoreType.DMA,
        ],
    )(_sc_body)
    out_sc = sc_call(x.reshape(b * srows * cols), idx).reshape(_SC_ROWS, k)

    out = jnp.concatenate([out_tc, out_sc], axis=0)
    return out.reshape(b, srows, k)


# speedup vs baseline: 4.7242x; 3.2781x over previous
import jax
import jax.numpy as jnp
from jax.experimental import pallas as pl
from jax.experimental.pallas import tpu as pltpu


_BLOCK_R = 512


def _select_body(x_ref, s_ref, o_ref):
    o_ref[...] = jnp.dot(
        x_ref[...], s_ref[...], preferred_element_type=jnp.float32
    )


def kernel(x, w):
    b, srows, cols = x.shape
    k = w.shape[0]
    xf = x.reshape(b * srows, cols)
    sel = (
        jax.lax.broadcasted_iota(jnp.int32, (cols, k), 0) == w[None, :]
    ).astype(jnp.float32)

    grid = (xf.shape[0] // _BLOCK_R,)
    out = pl.pallas_call(
        _select_body,
        grid=grid,
        in_specs=[
            pl.BlockSpec((_BLOCK_R, cols), lambda i: (i, 0)),
            pl.BlockSpec((cols, k), lambda i: (0, 0)),
        ],
        out_specs=pl.BlockSpec((_BLOCK_R, k), lambda i: (i, 0)),
        out_shape=jax.ShapeDtypeStruct((xf.shape[0], k), jnp.float32),
        compiler_params=pltpu.CompilerParams(
            vmem_limit_bytes=100 * 1024 * 1024,
        ),
    )(xf, sel)
    return out.reshape(b, srows, k)
